# baseline (device time: 33609 ns/iter reference)
import jax
import jax.numpy as jnp
from jax import lax
from jax.experimental import pallas as pl
from jax.experimental.pallas import tpu as pltpu

N_DEV = 4
E_PER_DEV = 4
N_EXPERTS = 16


def kernel(x, router_W, route_idx, expert_W, shared_W):
    T, D = x.shape
    H = shared_W.shape[1]

    def body(x_ref, rw_ref, idx_ref, ew_ref, sw_ref, out_ref,
             comm_ref, send_sems, recv_sems):
        my = lax.axis_index("i")
        left = lax.rem(my - 1 + N_DEV, N_DEV)
        right = lax.rem(my + 1, N_DEV)

        barrier_sem = pltpu.get_barrier_semaphore()
        for nbr in (left, right):
            pl.semaphore_signal(
                barrier_sem, inc=1,
                device_id=(nbr,), device_id_type=pl.DeviceIdType.MESH,
            )
        pl.semaphore_wait(barrier_sem, 2)

        xf = x_ref[...]
        scores = jnp.dot(xf, rw_ref[...],
                         preferred_element_type=jnp.float32)
        s_max = jnp.max(scores, axis=-1, keepdims=True)
        p = jnp.exp(scores - s_max)
        probs = p / jnp.sum(p, axis=-1, keepdims=True)

        idx = idx_ref[...]
        lanes = lax.broadcasted_iota(jnp.int32, (T, N_EXPERTS), 1)
        routed_p = jnp.sum(
            jnp.where(lanes == idx, probs, 0.0), axis=-1, keepdims=True
        )

        xb = xf.astype(jnp.bfloat16)
        partial = jnp.zeros((T, H), jnp.float32)
        for le in range(E_PER_DEV):
            e_global = my * E_PER_DEV + le
            w = jnp.where(idx == e_global, routed_p, 0.0)
            xs = (xb * w.astype(jnp.bfloat16))
            partial = partial + jnp.dot(
                xs, ew_ref[le].astype(jnp.bfloat16),
                preferred_element_type=jnp.float32,
            )

        comm_ref[0] = partial.astype(jnp.bfloat16)
        acc = partial
        for h in range(N_DEV - 1):
            rdma = pltpu.make_async_remote_copy(
                src_ref=comm_ref.at[h],
                dst_ref=comm_ref.at[h + 1],
                send_sem=send_sems.at[h],
                recv_sem=recv_sems.at[h],
                device_id=(right,),
                device_id_type=pl.DeviceIdType.MESH,
            )
            rdma.start()
            rdma.wait()
            acc = acc + comm_ref[h + 1].astype(jnp.float32)

        shared = jnp.dot(xb, sw_ref[...].astype(jnp.bfloat16),
                         preferred_element_type=jnp.float32)
        out_ref[...] = acc + shared

    return pl.pallas_call(
        body,
        out_shape=jax.ShapeDtypeStruct((T, H), jnp.float32),
        in_specs=[pl.BlockSpec(memory_space=pltpu.VMEM)] * 5,
        out_specs=pl.BlockSpec(memory_space=pltpu.VMEM),
        scratch_shapes=[
            pltpu.VMEM((N_DEV, T, H), jnp.bfloat16),
            pltpu.SemaphoreType.DMA((N_DEV - 1,)),
            pltpu.SemaphoreType.DMA((N_DEV - 1,)),
        ],
        compiler_params=pltpu.CompilerParams(collective_id=0),
    )(x, router_W, route_idx, expert_W, shared_W)


# device time: 24800 ns/iter; 1.3552x vs baseline; 1.3552x over previous
import jax
import jax.numpy as jnp
from jax import lax
from jax.experimental import pallas as pl
from jax.experimental.pallas import tpu as pltpu

N_DEV = 4
E_PER_DEV = 4
N_EXPERTS = 16


def kernel(x, router_W, route_idx, expert_W, shared_W):
    T, D = x.shape
    H = shared_W.shape[1]

    def body(x_ref, rw_ref, idx_ref, ew_ref, sw_ref, out_ref,
             sbuf_ref, comm_ref, send_sems, recv_sems):
        my = lax.axis_index("i")

        xf = x_ref[...]
        scores = jnp.dot(xf, rw_ref[...],
                         preferred_element_type=jnp.float32)
        s_max = jnp.max(scores, axis=-1, keepdims=True)
        p = jnp.exp(scores - s_max)
        probs = p / jnp.sum(p, axis=-1, keepdims=True)

        idx = idx_ref[...]
        lanes = lax.broadcasted_iota(jnp.int32, (T, N_EXPERTS), 1)
        routed_p = jnp.sum(
            jnp.where(lanes == idx, probs, 0.0), axis=-1, keepdims=True
        )

        xb = xf.astype(jnp.bfloat16)
        partial = jnp.zeros((T, H), jnp.float32)
        for le in range(E_PER_DEV):
            e_global = my * E_PER_DEV + le
            w = jnp.where(idx == e_global, routed_p, 0.0)
            xs = (xb * w.astype(jnp.bfloat16))
            partial = partial + jnp.dot(
                xs, ew_ref[le].astype(jnp.bfloat16),
                preferred_element_type=jnp.float32,
            )

        sbuf_ref[...] = partial.astype(jnp.bfloat16)

        barrier_sem = pltpu.get_barrier_semaphore()
        for o in range(1, N_DEV):
            peer = lax.rem(my + o, N_DEV)
            pl.semaphore_signal(
                barrier_sem, inc=1,
                device_id=(peer,), device_id_type=pl.DeviceIdType.MESH,
            )
        pl.semaphore_wait(barrier_sem, N_DEV - 1)

        rdmas = []
        for o in range(1, N_DEV):
            peer = lax.rem(my + o, N_DEV)
            slot = N_DEV - 1 - o
            rdma = pltpu.make_async_remote_copy(
                src_ref=sbuf_ref,
                dst_ref=comm_ref.at[slot],
                send_sem=send_sems.at[slot],
                recv_sem=recv_sems.at[slot],
                device_id=(peer,),
                device_id_type=pl.DeviceIdType.MESH,
            )
            rdma.start()
            rdmas.append(rdma)

        shared = jnp.dot(xb, sw_ref[...].astype(jnp.bfloat16),
                         preferred_element_type=jnp.float32)
        acc = partial + shared

        for rdma in rdmas:
            rdma.wait_recv()
        for slot in range(N_DEV - 1):
            acc = acc + comm_ref[slot].astype(jnp.float32)
        for rdma in rdmas:
            rdma.wait_send()

        out_ref[...] = acc

    return pl.pallas_call(
        body,
        out_shape=jax.ShapeDtypeStruct((T, H), jnp.float32),
        in_specs=[pl.BlockSpec(memory_space=pltpu.VMEM)] * 5,
        out_specs=pl.BlockSpec(memory_space=pltpu.VMEM),
        scratch_shapes=[
            pltpu.VMEM((T, H), jnp.bfloat16),
            pltpu.VMEM((N_DEV - 1, T, H), jnp.bfloat16),
            pltpu.SemaphoreType.DMA((N_DEV - 1,)),
            pltpu.SemaphoreType.DMA((N_DEV - 1,)),
        ],
        compiler_params=pltpu.CompilerParams(collective_id=0),
    )(x, router_W, route_idx, expert_W, shared_W)


# device time: 21149 ns/iter; 1.5892x vs baseline; 1.1726x over previous
import jax
import jax.numpy as jnp
from jax import lax
from jax.experimental import pallas as pl
from jax.experimental.pallas import tpu as pltpu

N_DEV = 4
E_PER_DEV = 4
N_EXPERTS = 16


def kernel(x, router_W, route_idx, expert_W, shared_W):
    T, D = x.shape
    H = shared_W.shape[1]

    Q = H // N_DEV

    def body(x_ref, rw_ref, idx_ref, ew_ref, sw_ref, out_ref,
             part_ref, pbuf_ref, rs_recv_ref, agbuf_ref, ag_recv_ref,
             rs_send_sems, rs_recv_sems, ag_send_sems, ag_recv_sems):
        my = lax.axis_index("i")

        xf = x_ref[...]
        scores = jnp.dot(xf, rw_ref[...],
                         preferred_element_type=jnp.float32)
        s_max = jnp.max(scores, axis=-1, keepdims=True)
        p = jnp.exp(scores - s_max)
        probs = p / jnp.sum(p, axis=-1, keepdims=True)

        idx = idx_ref[...]
        lanes = lax.broadcasted_iota(jnp.int32, (T, N_EXPERTS), 1)
        routed_p = jnp.sum(
            jnp.where(lanes == idx, probs, 0.0), axis=-1, keepdims=True
        )

        xb = xf.astype(jnp.bfloat16)
        partial = jnp.zeros((T, H), jnp.float32)
        for le in range(E_PER_DEV):
            e_global = my * E_PER_DEV + le
            w = jnp.where(idx == e_global, routed_p, 0.0)
            xs = (xb * w.astype(jnp.bfloat16))
            partial = partial + jnp.dot(
                xs, ew_ref[le].astype(jnp.bfloat16),
                preferred_element_type=jnp.float32,
            )

        part_ref[...] = partial.astype(jnp.bfloat16)
        for o in range(1, N_DEV):
            peer = lax.rem(my + o, N_DEV)
            pbuf_ref[o - 1] = part_ref[:, pl.ds(peer * Q, Q)]

        barrier_sem = pltpu.get_barrier_semaphore()
        for o in range(1, N_DEV):
            peer = lax.rem(my + o, N_DEV)
            pl.semaphore_signal(
                barrier_sem, inc=1,
                device_id=(peer,), device_id_type=pl.DeviceIdType.MESH,
            )
        pl.semaphore_wait(barrier_sem, N_DEV - 1)

        rs_rdmas = []
        for o in range(1, N_DEV):
            peer = lax.rem(my + o, N_DEV)
            slot = N_DEV - 1 - o
            rdma = pltpu.make_async_remote_copy(
                src_ref=pbuf_ref.at[o - 1],
                dst_ref=rs_recv_ref.at[slot],
                send_sem=rs_send_sems.at[slot],
                recv_sem=rs_recv_sems.at[slot],
                device_id=(peer,),
                device_id_type=pl.DeviceIdType.MESH,
            )
            rdma.start()
            rs_rdmas.append(rdma)

        shared = jnp.dot(xb, sw_ref[...].astype(jnp.bfloat16),
                         preferred_element_type=jnp.float32)
        out_ref[...] = partial + shared

        for rdma in rs_rdmas:
            rdma.wait_recv()
        red = out_ref[:, pl.ds(my * Q, Q)]
        for slot in range(N_DEV - 1):
            red = red + rs_recv_ref[slot].astype(jnp.float32)

        agbuf_ref[...] = red.astype(jnp.bfloat16)
        out_ref[:, pl.ds(my * Q, Q)] = red

        ag_rdmas = []
        for o in range(1, N_DEV):
            peer = lax.rem(my + o, N_DEV)
            slot = N_DEV - 1 - o
            rdma = pltpu.make_async_remote_copy(
                src_ref=agbuf_ref,
                dst_ref=ag_recv_ref.at[slot],
                send_sem=ag_send_sems.at[slot],
                recv_sem=ag_recv_sems.at[slot],
                device_id=(peer,),
                device_id_type=pl.DeviceIdType.MESH,
            )
            rdma.start()
            ag_rdmas.append(rdma)

        for rdma in ag_rdmas:
            rdma.wait_recv()
        for k in range(N_DEV - 1):
            s = lax.rem(my + k + 1, N_DEV)
            out_ref[:, pl.ds(s * Q, Q)] = ag_recv_ref[k].astype(jnp.float32)

        for rdma in rs_rdmas + ag_rdmas:
            rdma.wait_send()

    return pl.pallas_call(
        body,
        out_shape=jax.ShapeDtypeStruct((T, H), jnp.float32),
        in_specs=[pl.BlockSpec(memory_space=pltpu.VMEM)] * 5,
        out_specs=pl.BlockSpec(memory_space=pltpu.VMEM),
        scratch_shapes=[
            pltpu.VMEM((T, H), jnp.bfloat16),
            pltpu.VMEM((N_DEV - 1, T, H // N_DEV), jnp.bfloat16),
            pltpu.VMEM((N_DEV - 1, T, H // N_DEV), jnp.bfloat16),
            pltpu.VMEM((T, H // N_DEV), jnp.bfloat16),
            pltpu.VMEM((N_DEV - 1, T, H // N_DEV), jnp.bfloat16),
            pltpu.SemaphoreType.DMA((N_DEV - 1,)),
            pltpu.SemaphoreType.DMA((N_DEV - 1,)),
            pltpu.SemaphoreType.DMA((N_DEV - 1,)),
            pltpu.SemaphoreType.DMA((N_DEV - 1,)),
        ],
        compiler_params=pltpu.CompilerParams(collective_id=0),
    )(x, router_W, route_idx, expert_W, shared_W)


# device time: 19230 ns/iter; 1.7477x vs baseline; 1.0998x over previous
import jax
import jax.numpy as jnp
from jax import lax
from jax.experimental import pallas as pl
from jax.experimental.pallas import tpu as pltpu

N_DEV = 4
E_PER_DEV = 4
N_EXPERTS = 16


def kernel(x, router_W, route_idx, expert_W, shared_W):
    T, D = x.shape
    H = shared_W.shape[1]
    Q = H // N_DEV

    def body(x_ref, rw_ref, idx_ref, ew_ref, sw_ref, out_ref,
             part_ref, rs_recv_ref,
             rs_send_sems, rs_recv_sems, ag_send_sems, ag_recv_sems):
        my = lax.axis_index("i")

        barrier_sem = pltpu.get_barrier_semaphore()
        for o in range(1, N_DEV):
            peer = lax.rem(my + o, N_DEV)
            pl.semaphore_signal(
                barrier_sem, inc=1,
                device_id=(peer,), device_id_type=pl.DeviceIdType.MESH,
            )

        xf = x_ref[...]
        scores = jnp.dot(xf, rw_ref[...],
                         preferred_element_type=jnp.float32)
        s_max = jnp.max(scores, axis=-1, keepdims=True)
        p = jnp.exp(scores - s_max)
        probs = p / jnp.sum(p, axis=-1, keepdims=True)

        idx = idx_ref[...]
        lanes = lax.broadcasted_iota(jnp.int32, (T, N_EXPERTS), 1)
        routed_p = jnp.sum(
            jnp.where(lanes == idx, probs, 0.0), axis=-1, keepdims=True
        )

        xb = xf.astype(jnp.bfloat16)
        xs_parts = []
        for le in range(E_PER_DEV):
            e_global = my * E_PER_DEV + le
            w = jnp.where(idx == e_global, routed_p, 0.0)
            xs_parts.append(xb * w.astype(jnp.bfloat16))

        def partial_q(col):
            acc = None
            for le in range(E_PER_DEV):
                d = jnp.dot(xs_parts[le],
                            ew_ref[le, :, pl.ds(col, Q)].astype(jnp.bfloat16),
                            preferred_element_type=jnp.float32)
                acc = d if acc is None else acc + d
            return acc

        pl.semaphore_wait(barrier_sem, N_DEV - 1)

        TH = T // 2

        rs_rdmas = {}
        for o in (2, 1, 3):
            peer = lax.rem(my + o, N_DEV)
            slot = N_DEV - 1 - o
            part_ref[o - 1] = partial_q(peer * Q).astype(jnp.bfloat16)
            for h in range(2):
                rdma = pltpu.make_async_remote_copy(
                    src_ref=part_ref.at[o - 1, pl.ds(h * TH, TH)],
                    dst_ref=rs_recv_ref.at[slot, pl.ds(h * TH, TH)],
                    send_sem=rs_send_sems.at[h, slot],
                    recv_sem=rs_recv_sems.at[h, slot],
                    device_id=(peer,),
                    device_id_type=pl.DeviceIdType.MESH,
                )
                rdma.start()
                rs_rdmas[(h, o)] = rdma

        red = partial_q(my * Q)
        red = red + jnp.dot(
            xb, sw_ref[:, pl.ds(my * Q, Q)].astype(jnp.bfloat16),
            preferred_element_type=jnp.float32,
        )

        ag_rdmas = []
        for h in range(2):
            red_h = red[h * TH:(h + 1) * TH, :]
            for o in (1, 3, 2):
                slot = N_DEV - 1 - o
                rs_rdmas[(h, o)].wait_recv()
                red_h = red_h + rs_recv_ref[
                    slot, pl.ds(h * TH, TH)].astype(jnp.float32)
            out_ref[pl.ds(h * TH, TH), pl.ds(my * Q, Q)] = (
                red_h.astype(jnp.bfloat16))
            for o in (2, 1, 3):
                peer = lax.rem(my + o, N_DEV)
                slot = N_DEV - 1 - o
                rdma = pltpu.make_async_remote_copy(
                    src_ref=out_ref.at[pl.ds(h * TH, TH), pl.ds(my * Q, Q)],
                    dst_ref=out_ref.at[pl.ds(h * TH, TH), pl.ds(my * Q, Q)],
                    send_sem=ag_send_sems.at[h, slot],
                    recv_sem=ag_recv_sems.at[h, slot],
                    device_id=(peer,),
                    device_id_type=pl.DeviceIdType.MESH,
                )
                rdma.start()
                ag_rdmas.append(rdma)

        for rdma in ag_rdmas:
            rdma.wait_recv()
        for rdma in list(rs_rdmas.values()) + ag_rdmas:
            rdma.wait_send()

    return pl.pallas_call(
        body,
        out_shape=jax.ShapeDtypeStruct((T, H), jnp.bfloat16),
        in_specs=[pl.BlockSpec(memory_space=pltpu.VMEM)] * 5,
        out_specs=pl.BlockSpec(memory_space=pltpu.VMEM),
        scratch_shapes=[
            pltpu.VMEM((N_DEV - 1, T, Q), jnp.bfloat16),
            pltpu.VMEM((N_DEV - 1, T, Q), jnp.bfloat16),
            pltpu.SemaphoreType.DMA((2, N_DEV - 1)),
            pltpu.SemaphoreType.DMA((2, N_DEV - 1)),
            pltpu.SemaphoreType.DMA((2, N_DEV - 1)),
            pltpu.SemaphoreType.DMA((2, N_DEV - 1)),
        ],
        compiler_params=pltpu.CompilerParams(collective_id=0),
    )(x, router_W, route_idx, expert_W, shared_W)
